# baseline (speedup 1.0000x reference)
import functools

import jax
import jax.numpy as jnp
from jax import lax
from jax.experimental import pallas as pl
from jax.experimental.pallas import tpu as pltpu
from jax.experimental.pallas import tpu_sc as plsc

N = 10000
E = 320000
D = 128
NC = 2
NS = 16
CH = 128
NP = 10240
RPT = NP // NS
NCHUNK = 80
EPT = NCHUNK * CH
EH = NS * EPT
EPP = NC * EH
EROWS = EPP // CH
CPT = EH // CH
G = 16
NG = NCHUNK // G
NB = 2

_mesh = plsc.VectorSubcoreMesh(core_axis_name="c", subcore_axis_name="s")


@functools.partial(
    pl.kernel,
    out_type=jax.ShapeDtypeStruct((NC, NP), jnp.float32),
    mesh=_mesh,
    scratch_types=[
        pltpu.VMEM((NCHUNK, CH), jnp.int32),
        pltpu.VMEM((CH,), jnp.float32),
        pltpu.VMEM((RPT,), jnp.float32),
        pltpu.VMEM_SHARED((NP,), jnp.float32),
        pltpu.SemaphoreType.DMA,
    ],
)
def _deg_call(src_hbm, out_hbm, idx_v, ones_v, zbuf_v, deg_sh, sem):
    cc = lax.axis_index("c")
    ss = lax.axis_index("s")

    def fill(i, _):
        zbuf_v[pl.ds(i * 16, 16)] = jnp.zeros((16,), jnp.float32)
        return 0
    lax.fori_loop(0, RPT // 16, fill, 0)

    def fill1(i, _):
        ones_v[pl.ds(i * 16, 16)] = jnp.ones((16,), jnp.float32)
        return 0
    lax.fori_loop(0, CH // 16, fill1, 0)

    rowbase = cc * CPT + ss * NCHUNK
    pltpu.sync_copy(src_hbm.at[pl.ds(rowbase, NCHUNK)], idx_v)
    pltpu.sync_copy(zbuf_v, deg_sh.at[pl.ds(ss * RPT, RPT)])
    plsc.subcore_barrier()

    def fire(j, _):
        pltpu.async_copy(ones_v, deg_sh.at[idx_v.at[j]], sem, add=True)
        return 0
    lax.fori_loop(0, NCHUNK, fire, 0)

    def drain(j, _):
        pltpu.make_async_copy(ones_v, deg_sh.at[idx_v.at[0]], sem).wait()
        return 0
    lax.fori_loop(0, NCHUNK, drain, 0)

    plsc.subcore_barrier()
    pltpu.sync_copy(deg_sh.at[pl.ds(ss * RPT, RPT)],
                    out_hbm.at[cc, pl.ds(ss * RPT, RPT)])


_BR = 2048

def _mm_body(x_ref, w_ref, d0_ref, d1_ref, y_ref):
    deg = d0_ref[...] + d1_ref[...]
    dinv = jnp.where(deg > 0.0, 1.0 / deg, 0.0)
    xw = jnp.dot(x_ref[...], w_ref[...], preferred_element_type=jnp.float32)
    y_ref[...] = xw * dinv


_mm_call = pl.pallas_call(
    _mm_body,
    grid=(NP // _BR,),
    in_specs=[
        pl.BlockSpec((_BR, D), lambda i: (i, 0)),
        pl.BlockSpec((D, D), lambda i: (0, 0)),
        pl.BlockSpec((_BR, 1), lambda i: (i, 0)),
        pl.BlockSpec((_BR, 1), lambda i: (i, 0)),
    ],
    out_specs=pl.BlockSpec((_BR, D), lambda i: (i, 0)),
    out_shape=jax.ShapeDtypeStruct((NP, D), jnp.float32),
)


@functools.partial(
    pl.kernel,
    out_type=jax.ShapeDtypeStruct((NC, NP, D), jnp.float32),
    mesh=_mesh,
    scratch_types=[
        pltpu.VMEM((G, CH), jnp.int32),
        pltpu.VMEM((G, CH), jnp.int32),
        pltpu.VMEM((NB, CH, D), jnp.float32),
        pltpu.VMEM_SHARED((NP, D), jnp.float32),
        pltpu.SemaphoreType.DMA,
        pltpu.SemaphoreType.DMA,
    ],
)
def _gs_call(y_hbm, src_hbm, dst_hbm, out_hbm, sidx_v, didx_v, rows_v,
             acc_sh, sem0, sem1):
    cc = lax.axis_index("c")
    ss = lax.axis_index("s")
    sems = [sem0, sem1]

    def fill(i, _):
        r = i // (D // 16)
        c = i % (D // 16)
        rows_v[0, r, pl.ds(c * 16, 16)] = jnp.zeros((16,), jnp.float32)
        return 0
    lax.fori_loop(0, CH * (D // 16), fill, 0)

    def zcopy(k, _):
        pltpu.sync_copy(rows_v.at[0], acc_sh.at[pl.ds(ss * RPT + k * CH, CH)])
        return 0
    lax.fori_loop(0, RPT // CH, zcopy, 0)
    plsc.subcore_barrier()

    rowbase = cc * CPT + ss * NCHUNK

    def issue_gather(i, k):
        pltpu.async_copy(y_hbm.at[sidx_v.at[i]], rows_v.at[k], sems[k])

    def wait_gather(i, k):
        pltpu.make_async_copy(y_hbm.at[sidx_v.at[i]], rows_v.at[k],
                              sems[k]).wait()

    def group(g, _):
        pltpu.sync_copy(src_hbm.at[pl.ds(rowbase + g * G, G)], sidx_v)
        pltpu.sync_copy(dst_hbm.at[pl.ds(rowbase + g * G, G)], didx_v)
        for k in range(NB):
            issue_gather(k, k)

        def step(t, _):
            for k in range(NB):
                i = t * NB + k
                wait_gather(i, k)
                pltpu.sync_copy(rows_v.at[k], acc_sh.at[didx_v.at[i]],
                                add=True)

                @pl.when(t < G // NB - 1)
                def _():
                    issue_gather(i + NB, k)
            return 0
        lax.fori_loop(0, G // NB, step, 0)
        return 0
    lax.fori_loop(0, NG, group, 0)

    plsc.subcore_barrier()
    pltpu.sync_copy(acc_sh.at[pl.ds(ss * RPT, RPT)],
                    out_hbm.at[cc, pl.ds(ss * RPT, RPT)])


_BO = 2000

def _comb_body(p_ref, b_ref, o_ref):
    o_ref[...] = p_ref[0] + p_ref[1] + b_ref[...]


_comb_call = pl.pallas_call(
    _comb_body,
    grid=(N // _BO,),
    in_specs=[
        pl.BlockSpec((NC, _BO, D), lambda i: (0, i, 0)),
        pl.BlockSpec((1, D), lambda i: (0, 0)),
    ],
    out_specs=pl.BlockSpec((_BO, D), lambda i: (i, 0)),
    out_shape=jax.ShapeDtypeStruct((N, D), jnp.float32),
)


def kernel(x, edge_index, W, b):
    src = edge_index[0].astype(jnp.int32)
    dst = edge_index[1].astype(jnp.int32)
    pad = jnp.full((EPP - E,), N, dtype=jnp.int32)
    src_p = jnp.concatenate([src, pad]).reshape(EROWS, CH)
    dst_p = jnp.concatenate([dst, pad]).reshape(EROWS, CH)
    x_p = jnp.concatenate([x, jnp.zeros((NP - N, D), x.dtype)])

    degs = _deg_call(src_p)
    d0 = degs[0].reshape(NP, 1)
    d1 = degs[1].reshape(NP, 1)
    y = _mm_call(x_p, W, d0, d1)
    parts = _gs_call(y, src_p, dst_p)
    return _comb_call(parts, b.reshape(1, D))


# speedup vs baseline: 3.5134x; 3.5134x over previous
import functools

import jax
import jax.numpy as jnp
from jax import lax
from jax.experimental import pallas as pl
from jax.experimental.pallas import tpu as pltpu
from jax.experimental.pallas import tpu_sc as plsc

N = 10000
E = 320000
D = 128
NC = 2
NS = 16
CH = 128
NP = 10240
RPT = NP // NS
NCHUNK = 80
EPT = NCHUNK * CH
EH = NS * EPT
EPP = NC * EH
EROWS = EPP // CH
CPT = EH // CH
G = 16
NG = NCHUNK // G
NB = 2

_mesh = plsc.VectorSubcoreMesh(core_axis_name="c", subcore_axis_name="s")


@functools.partial(
    pl.kernel,
    out_type=jax.ShapeDtypeStruct((NC, NP), jnp.float32),
    mesh=_mesh,
    scratch_types=[
        pltpu.VMEM((NCHUNK, CH), jnp.int32),
        pltpu.VMEM((CH,), jnp.float32),
        pltpu.VMEM((RPT,), jnp.float32),
        pltpu.VMEM_SHARED((NP,), jnp.float32),
        pltpu.SemaphoreType.DMA,
    ],
)
def _deg_call(src_hbm, out_hbm, idx_v, ones_v, zbuf_v, deg_sh, sem):
    cc = lax.axis_index("c")
    ss = lax.axis_index("s")

    def fill(i, _):
        zbuf_v[pl.ds(i * 16, 16)] = jnp.zeros((16,), jnp.float32)
        return 0
    lax.fori_loop(0, RPT // 16, fill, 0)

    def fill1(i, _):
        ones_v[pl.ds(i * 16, 16)] = jnp.ones((16,), jnp.float32)
        return 0
    lax.fori_loop(0, CH // 16, fill1, 0)

    rowbase = cc * CPT + ss * NCHUNK
    pltpu.sync_copy(src_hbm.at[pl.ds(rowbase, NCHUNK)], idx_v)
    pltpu.sync_copy(zbuf_v, deg_sh.at[pl.ds(ss * RPT, RPT)])
    plsc.subcore_barrier()

    def fire(j, _):
        pltpu.async_copy(ones_v, deg_sh.at[idx_v.at[j]], sem, add=True)
        return 0
    lax.fori_loop(0, NCHUNK, fire, 0)

    def drain(j, _):
        pltpu.make_async_copy(ones_v, deg_sh.at[idx_v.at[0]], sem).wait()
        return 0
    lax.fori_loop(0, NCHUNK, drain, 0)

    plsc.subcore_barrier()
    pltpu.sync_copy(deg_sh.at[pl.ds(ss * RPT, RPT)],
                    out_hbm.at[cc, pl.ds(ss * RPT, RPT)])


_BR = 2048

def _mm_body(x_ref, w_ref, d0_ref, d1_ref, y_ref):
    deg = d0_ref[...] + d1_ref[...]
    dinv = jnp.where(deg > 0.0, 1.0 / deg, 0.0)
    xw = jnp.dot(x_ref[...], w_ref[...], preferred_element_type=jnp.float32)
    y_ref[...] = xw * dinv


_mm_call = pl.pallas_call(
    _mm_body,
    grid=(NP // _BR,),
    in_specs=[
        pl.BlockSpec((_BR, D), lambda i: (i, 0)),
        pl.BlockSpec((D, D), lambda i: (0, 0)),
        pl.BlockSpec((_BR, 1), lambda i: (i, 0)),
        pl.BlockSpec((_BR, 1), lambda i: (i, 0)),
    ],
    out_specs=pl.BlockSpec((_BR, D), lambda i: (i, 0)),
    out_shape=jax.ShapeDtypeStruct((NP, D), jnp.float32),
)


@functools.partial(
    pl.kernel,
    out_type=jax.ShapeDtypeStruct((NC, NP, D), jnp.float32),
    mesh=_mesh,
    scratch_types=[
        pltpu.VMEM((G, CH), jnp.int32),
        pltpu.VMEM((G, CH), jnp.int32),
        pltpu.VMEM((NB, CH, D), jnp.float32),
        pltpu.VMEM_SHARED((NP, D), jnp.float32),
        pltpu.SemaphoreType.DMA,
        pltpu.SemaphoreType.DMA,
    ],
)
def _gs_call(y_hbm, src_hbm, dst_hbm, out_hbm, sidx_v, didx_v, rows_v,
             acc_sh, sem0, sem1):
    cc = lax.axis_index("c")
    ss = lax.axis_index("s")
    sems = [sem0, sem1]

    def fill(i, _):
        r = i // (D // 16)
        c = i % (D // 16)
        rows_v[0, r, pl.ds(c * 16, 16)] = jnp.zeros((16,), jnp.float32)
        return 0
    lax.fori_loop(0, CH * (D // 16), fill, 0)

    def zcopy(k, _):
        pltpu.sync_copy(rows_v.at[0], acc_sh.at[pl.ds(ss * RPT + k * CH, CH)])
        return 0
    lax.fori_loop(0, RPT // CH, zcopy, 0)
    plsc.subcore_barrier()

    rowbase = cc * CPT + ss * NCHUNK

    def issue_gather(i, k):
        pltpu.async_copy(y_hbm.at[sidx_v.at[i]], rows_v.at[k], sems[k])

    def wait_gather(i, k):
        pltpu.make_async_copy(y_hbm.at[sidx_v.at[i]], rows_v.at[k],
                              sems[k]).wait()

    def group(g, _):
        pltpu.sync_copy(src_hbm.at[pl.ds(rowbase + g * G, G)], sidx_v)
        pltpu.sync_copy(dst_hbm.at[pl.ds(rowbase + g * G, G)], didx_v)
        for k in range(NB):
            issue_gather(k, k)

        def step(t, _):
            for k in range(NB):
                i = t * NB + k
                wait_gather(i, k)
                pltpu.sync_copy(rows_v.at[k], acc_sh.at[didx_v.at[i]],
                                add=True)

                @pl.when(t < G // NB - 1)
                def _():
                    issue_gather(i + NB, k)
            return 0
        lax.fori_loop(0, G // NB, step, 0)
        return 0
    lax.fori_loop(0, NG, group, 0)

    plsc.subcore_barrier()
    pltpu.sync_copy(acc_sh.at[pl.ds(ss * RPT, RPT)],
                    out_hbm.at[cc, pl.ds(ss * RPT, RPT)])


_BO = 2000

def _comb_body(p_ref, b_ref, o_ref):
    o_ref[...] = p_ref[0] + p_ref[1] + b_ref[...]


_comb_call = pl.pallas_call(
    _comb_body,
    grid=(N // _BO,),
    in_specs=[
        pl.BlockSpec((NC, _BO, D), lambda i: (0, i, 0)),
        pl.BlockSpec((1, D), lambda i: (0, 0)),
    ],
    out_specs=pl.BlockSpec((_BO, D), lambda i: (i, 0)),
    out_shape=jax.ShapeDtypeStruct((N, D), jnp.float32),
)


def kernel(x, edge_index, W, b):
    src = edge_index[0].astype(jnp.int32)
    dst = edge_index[1].astype(jnp.int32)
    pad = N + jnp.arange(EPP - E, dtype=jnp.int32) % (NP - N)
    src_p = jnp.concatenate([src, pad]).reshape(EROWS, CH)
    dst_p = jnp.concatenate([dst, pad]).reshape(EROWS, CH)
    x_p = jnp.concatenate([x, jnp.zeros((NP - N, D), x.dtype)])

    degs = _deg_call(src_p)
    d0 = degs[0].reshape(NP, 1)
    d1 = degs[1].reshape(NP, 1)
    y = _mm_call(x_p, W, d0, d1)
    parts = _gs_call(y, src_p, dst_p)
    return _comb_call(parts, b.reshape(1, D))


# speedup vs baseline: 3.7691x; 1.0728x over previous
import functools

import jax
import jax.numpy as jnp
from jax import lax
from jax.experimental import pallas as pl
from jax.experimental.pallas import tpu as pltpu
from jax.experimental.pallas import tpu_sc as plsc

N = 10000
E = 320000
D = 128
NC = 2
NS = 16
CH = 128
NP = 10240
RPT = NP // NS
NCHUNK = 80
EPT = NCHUNK * CH
EH = NS * EPT
EPP = NC * EH
EROWS = EPP // CH
CPT = EH // CH
G = 16
NG = NCHUNK // G
NB = 2

_mesh = plsc.VectorSubcoreMesh(core_axis_name="c", subcore_axis_name="s")


@functools.partial(
    pl.kernel,
    out_type=jax.ShapeDtypeStruct((NC, NP), jnp.float32),
    mesh=_mesh,
    scratch_types=[
        pltpu.VMEM((NCHUNK, CH), jnp.int32),
        pltpu.VMEM((CH,), jnp.float32),
        pltpu.VMEM((RPT,), jnp.float32),
        pltpu.VMEM_SHARED((NP,), jnp.float32),
        pltpu.SemaphoreType.DMA,
    ],
)
def _deg_call(src_hbm, out_hbm, idx_v, ones_v, zbuf_v, deg_sh, sem):
    cc = lax.axis_index("c")
    ss = lax.axis_index("s")

    def fill(i, _):
        zbuf_v[pl.ds(i * 16, 16)] = jnp.zeros((16,), jnp.float32)
        return 0
    lax.fori_loop(0, RPT // 16, fill, 0)

    def fill1(i, _):
        ones_v[pl.ds(i * 16, 16)] = jnp.ones((16,), jnp.float32)
        return 0
    lax.fori_loop(0, CH // 16, fill1, 0)

    rowbase = cc * CPT + ss * NCHUNK
    pltpu.sync_copy(src_hbm.at[pl.ds(rowbase, NCHUNK)], idx_v)
    pltpu.sync_copy(zbuf_v, deg_sh.at[pl.ds(ss * RPT, RPT)])
    plsc.subcore_barrier()

    def fire(j, _):
        pltpu.async_copy(ones_v, deg_sh.at[idx_v.at[j]], sem, add=True)
        return 0
    lax.fori_loop(0, NCHUNK, fire, 0)

    def drain(j, _):
        pltpu.make_async_copy(ones_v, deg_sh.at[idx_v.at[0]], sem).wait()
        return 0
    lax.fori_loop(0, NCHUNK, drain, 0)

    plsc.subcore_barrier()
    pltpu.sync_copy(deg_sh.at[pl.ds(ss * RPT, RPT)],
                    out_hbm.at[cc, pl.ds(ss * RPT, RPT)])


_BR = 2048

def _mm_body(x_ref, w_ref, d0_ref, d1_ref, y_ref):
    deg = d0_ref[...] + d1_ref[...]
    dinv = jnp.where(deg > 0.0, 1.0 / deg, 0.0)
    xw = jnp.dot(x_ref[...], w_ref[...], preferred_element_type=jnp.float32)
    y_ref[...] = xw * dinv


_mm_call = pl.pallas_call(
    _mm_body,
    grid=(NP // _BR,),
    in_specs=[
        pl.BlockSpec((_BR, D), lambda i: (i, 0)),
        pl.BlockSpec((D, D), lambda i: (0, 0)),
        pl.BlockSpec((_BR, 1), lambda i: (i, 0)),
        pl.BlockSpec((_BR, 1), lambda i: (i, 0)),
    ],
    out_specs=pl.BlockSpec((_BR, D), lambda i: (i, 0)),
    out_shape=jax.ShapeDtypeStruct((NP, D), jnp.float32),
)


@functools.partial(
    pl.kernel,
    out_type=jax.ShapeDtypeStruct((NC, NP, D), jnp.float32),
    mesh=_mesh,
    scratch_types=[
        pltpu.VMEM((2 * G, CH), jnp.int32),
        pltpu.VMEM((2 * G, CH), jnp.int32),
        pltpu.VMEM((NB, CH, D), jnp.float32),
        pltpu.VMEM_SHARED((NP, D), jnp.float32),
        pltpu.SemaphoreType.DMA,
        pltpu.SemaphoreType.DMA,
        pltpu.SemaphoreType.DMA,
    ],
)
def _gs_call(y_hbm, src_hbm, dst_hbm, out_hbm, sidx_v, didx_v, rows_v,
             acc_sh, sem0, sem1, isem):
    cc = lax.axis_index("c")
    ss = lax.axis_index("s")
    sems = [sem0, sem1]
    rowbase = cc * CPT + ss * NCHUNK

    def fill(i, _):
        r = i // (D // 16)
        c = i % (D // 16)
        rows_v[0, r, pl.ds(c * 16, 16)] = jnp.zeros((16,), jnp.float32)
        return 0
    lax.fori_loop(0, CH * (D // 16), fill, 0)

    def zcopy(k, _):
        pltpu.async_copy(rows_v.at[0],
                         acc_sh.at[pl.ds(ss * RPT + k * CH, CH)], isem)
        return 0
    lax.fori_loop(0, RPT // CH, zcopy, 0)
    pltpu.sync_copy(src_hbm.at[pl.ds(rowbase, G)], sidx_v.at[pl.ds(0, G)])
    pltpu.sync_copy(dst_hbm.at[pl.ds(rowbase, G)], didx_v.at[pl.ds(0, G)])

    def zdrain(k, _):
        pltpu.make_async_copy(rows_v.at[0],
                              acc_sh.at[pl.ds(ss * RPT, CH)], isem).wait()
        return 0
    lax.fori_loop(0, RPT // CH, zdrain, 0)
    plsc.subcore_barrier()

    def issue_gather(brow, k):
        pltpu.async_copy(y_hbm.at[sidx_v.at[brow]], rows_v.at[k], sems[k])

    def wait_gather(k):
        pltpu.make_async_copy(y_hbm.at[sidx_v.at[0]], rows_v.at[k],
                              sems[k]).wait()

    for k in range(NB):
        issue_gather(k, k)

    def group(g, _):
        gb = (g % 2) * G
        nb_ = ((g + 1) % 2) * G

        @pl.when(g + 1 < NG)
        def _():
            nxt = rowbase + (g + 1) * G
            pltpu.async_copy(src_hbm.at[pl.ds(nxt, G)],
                             sidx_v.at[pl.ds(nb_, G)], isem)
            pltpu.async_copy(dst_hbm.at[pl.ds(nxt, G)],
                             didx_v.at[pl.ds(nb_, G)], isem)

        def step(t, _):
            for k in range(NB):
                i = t * NB + k
                wait_gather(k)
                pltpu.sync_copy(rows_v.at[k], acc_sh.at[didx_v.at[gb + i]],
                                add=True)
                issue_gather(gb + i + NB, k)
            return 0
        lax.fori_loop(0, G // NB - 1, step, 0)

        for k in range(NB):
            i = G - NB + k
            wait_gather(k)
            pltpu.sync_copy(rows_v.at[k], acc_sh.at[didx_v.at[gb + i]],
                            add=True)
            if k == 0:
                @pl.when(g + 1 < NG)
                def _():
                    pltpu.make_async_copy(
                        src_hbm.at[pl.ds(rowbase, G)],
                        sidx_v.at[pl.ds(nb_, G)], isem).wait()
                    pltpu.make_async_copy(
                        dst_hbm.at[pl.ds(rowbase, G)],
                        didx_v.at[pl.ds(nb_, G)], isem).wait()

            @pl.when(g + 1 < NG)
            def _():
                issue_gather(nb_ + i + NB - G, k)
        return 0
    lax.fori_loop(0, NG, group, 0)

    plsc.subcore_barrier()
    pltpu.sync_copy(acc_sh.at[pl.ds(ss * RPT, RPT)],
                    out_hbm.at[cc, pl.ds(ss * RPT, RPT)])


_BO = 2000

def _comb_body(p_ref, b_ref, o_ref):
    o_ref[...] = p_ref[0] + p_ref[1] + b_ref[...]


_comb_call = pl.pallas_call(
    _comb_body,
    grid=(N // _BO,),
    in_specs=[
        pl.BlockSpec((NC, _BO, D), lambda i: (0, i, 0)),
        pl.BlockSpec((1, D), lambda i: (0, 0)),
    ],
    out_specs=pl.BlockSpec((_BO, D), lambda i: (i, 0)),
    out_shape=jax.ShapeDtypeStruct((N, D), jnp.float32),
)


def kernel(x, edge_index, W, b):
    src = edge_index[0].astype(jnp.int32)
    dst = edge_index[1].astype(jnp.int32)
    pad = N + jnp.arange(EPP - E, dtype=jnp.int32) % (NP - N)
    src_p = jnp.concatenate([src, pad]).reshape(EROWS, CH)
    dst_p = jnp.concatenate([dst, pad]).reshape(EROWS, CH)
    x_p = jnp.concatenate([x, jnp.zeros((NP - N, D), x.dtype)])

    degs = _deg_call(src_p)
    d0 = degs[0].reshape(NP, 1)
    d1 = degs[1].reshape(NP, 1)
    y = _mm_call(x_p, W, d0, d1)
    parts = _gs_call(y, src_p, dst_p)
    return _comb_call(parts, b.reshape(1, D))
